# baseline (speedup 1.0000x reference)
import functools

import jax
import jax.numpy as jnp
from jax import lax
from jax.experimental import pallas as pl
from jax.experimental.pallas import tpu as pltpu
from jax.experimental.pallas import tpu_sc as plsc

_NUM_EMBEDDINGS = 1000000
_DIM = 32
_BATCH = 16384
_ROWS_PER_BIG = 128 // _DIM
_BIG_ROWS = _NUM_EMBEDDINGS // _ROWS_PER_BIG

_info = plsc.get_sparse_core_info()
_NC, _NS, _L = _info.num_cores, _info.num_subcores, _info.num_lanes
_NW = _NC * _NS
_BPW = _BATCH // _NW
_CHUNKS = _BPW // 128

_mesh = plsc.VectorSubcoreMesh(core_axis_name="c", subcore_axis_name="s")


@functools.partial(
    pl.kernel,
    mesh=_mesh,
    out_type=jax.ShapeDtypeStruct((_BATCH // _ROWS_PER_BIG, 128), jnp.float32),
    scratch_types=[
        pltpu.VMEM((_CHUNKS, 128), jnp.int32),
        pltpu.VMEM((_CHUNKS, 128), jnp.int32),
        pltpu.VMEM((_BPW, 128), jnp.float32),
        pltpu.VMEM((_BPW // _ROWS_PER_BIG, 128), jnp.float32),
        pltpu.SemaphoreType.DMA,
    ],
    compiler_params=pltpu.CompilerParams(needs_layout_passes=False),
)
def _gather_kernel(div_hbm, off_hbm, table_hbm, out_hbm,
                   div_v, off_v, big_v, out_v, sem):
    wid = lax.axis_index("s") * _NC + lax.axis_index("c")
    pltpu.sync_copy(div_hbm.at[pl.ds(wid * _CHUNKS, _CHUNKS)], div_v)
    pltpu.sync_copy(off_hbm.at[pl.ds(wid * _CHUNKS, _CHUNKS)], off_v)
    copies = [
        pltpu.async_copy(table_hbm.at[div_v.at[j]],
                         big_v.at[pl.ds(j * 128, 128)], sem)
        for j in range(_CHUNKS)
    ]
    for c in copies:
        c.wait()
    lane = lax.iota(jnp.int32, _L)
    for g in range(_BPW // _L):
        rows = lane + g * _L
        offs = off_v[g // 8, pl.ds((g % 8) * _L, _L)]
        q = lax.shift_right_logical(rows, 2)
        jbase = lax.shift_left(lax.bitwise_and(rows, 3), 5)
        for c in range(_DIM):
            val = plsc.load_gather(big_v, [rows, offs + c])
            plsc.store_scatter(out_v, [q, jbase + c], val)
    pltpu.sync_copy(out_v, out_hbm.at[pl.ds(wid * (_BPW // _ROWS_PER_BIG),
                                            _BPW // _ROWS_PER_BIG)])


def kernel(input_index, embeds):
    idx = input_index.astype(jnp.int32)
    div = (idx // _ROWS_PER_BIG).reshape(_NW * _CHUNKS, 128)
    off = ((idx % _ROWS_PER_BIG) * _DIM).reshape(_NW * _CHUNKS, 128)
    table = embeds.reshape(_BIG_ROWS, 128)
    out = _gather_kernel(div, off, table)
    return out.reshape(_BATCH, _DIM)


# speedup vs baseline: 1.7006x; 1.7006x over previous
import functools

import jax
import jax.numpy as jnp
from jax import lax
from jax.experimental import pallas as pl
from jax.experimental.pallas import tpu as pltpu
from jax.experimental.pallas import tpu_sc as plsc

_NUM_EMBEDDINGS = 1000000
_DIM = 32
_BATCH = 16384

_info = plsc.get_sparse_core_info()
_NC, _NS, _L = _info.num_cores, _info.num_subcores, _info.num_lanes
_NW = _NC * _NS
_BPW = _BATCH // _NW
_G = _BPW // _L

_mesh = plsc.VectorSubcoreMesh(core_axis_name="c", subcore_axis_name="s")


@functools.partial(
    pl.kernel,
    mesh=_mesh,
    out_type=jax.ShapeDtypeStruct((_BATCH, _DIM), jnp.float32),
    scratch_types=[
        pltpu.VMEM((_BPW,), jnp.int32),
        pltpu.VMEM((_BPW, _DIM), jnp.float32),
        pltpu.SemaphoreType.DMA,
    ],
    compiler_params=pltpu.CompilerParams(needs_layout_passes=False),
)
def _gather_kernel(idx_hbm, table_hbm, out_hbm, idx_v, rows_v, sem):
    wid = lax.axis_index("s") * _NC + lax.axis_index("c")
    base = wid * _BPW
    pltpu.sync_copy(idx_hbm.at[pl.ds(base, _BPW)], idx_v)

    lane = lax.iota(jnp.int32, _L)

    def fire_group(g):
        vec = idx_v[pl.ds(g * _L, _L)]
        for l in range(_L):
            rv = jnp.sum(jnp.where(lane == l, vec, 0))
            pltpu.async_copy(table_hbm.at[pl.ds(rv, 1)],
                             rows_v.at[pl.ds(g * _L + l, 1)], sem)

    def drain_group():
        for _ in range(_L):
            pltpu.make_async_copy(table_hbm.at[pl.ds(0, 1)],
                                  rows_v.at[pl.ds(0, 1)], sem).wait()

    fire_group(0)

    def body(g, carry):
        fire_group(g)
        drain_group()
        return carry

    lax.fori_loop(1, _G, body, 0)
    drain_group()

    pltpu.sync_copy(rows_v, out_hbm.at[pl.ds(base, _BPW)])


def kernel(input_index, embeds):
    return _gather_kernel(input_index.astype(jnp.int32), embeds)
